# baseline (speedup 1.0000x reference)
import functools

import jax
import jax.numpy as jnp
from jax import lax
from jax.experimental import pallas as pl
from jax.experimental.pallas import tpu as pltpu
from jax.experimental.pallas import tpu_sc as plsc

_VOCAB = 8192
_C = 64
_B, _T, _H, _W = 4, 8, 24, 24
_N = _B * _T * _H * _W
_TOK = _T * _H * _W
_TN = 512
_NB = _N // _TN
_BETA = 0.25
_RESI = 0.5


def _s1_body(q_ref, emb_ref, idx_ref, cb_ref):
    i = pl.program_id(0)

    @pl.when(i == 0)
    def _init():
        e = emb_ref[...]
        n = jnp.sqrt(jnp.sum(e * e, axis=1, keepdims=True))
        cb_ref[...] = e / jnp.maximum(n, 1e-12)

    q = q_ref[0].T
    qn = jnp.sqrt(jnp.sum(q * q, axis=1, keepdims=True))
    q = q / jnp.maximum(qn, 1e-12)
    lane = lax.broadcasted_iota(jnp.int32, (1, 128), 1).astype(jnp.float32)
    s = lax.dot_general(q, cb_ref[...], (((1,), (1,)), ((), ())),
                        preferred_element_type=jnp.float32)
    for g in range(_TN // 64):
        rows = slice(g * 64, (g + 1) * 64)
        bestv = jnp.full((64, 128), -jnp.inf, jnp.float32)
        besti = jnp.zeros((64, 128), jnp.float32)
        for j in range(_VOCAB // 128):
            blk = s[rows, j * 128:(j + 1) * 128]
            gt = blk > bestv
            besti = jnp.where(gt, float(j), besti)
            bestv = jnp.maximum(blk, bestv)
        m = jnp.max(bestv, axis=1, keepdims=True)
        cand = jnp.where(bestv == m, besti * 128.0 + lane, 1e9)
        idx = jnp.min(cand, axis=1)
        idx_ref[0, 0, rows] = idx.astype(jnp.int32)


_CPB = _TOK // _TN


def _stage1(f_nat, emb):
    return pl.pallas_call(
        _s1_body,
        grid=(_NB,),
        in_specs=[
            pl.BlockSpec((1, _C, _TN), lambda i: (i // _CPB, 0, i % _CPB)),
            pl.BlockSpec((_VOCAB, _C), lambda i: (0, 0)),
        ],
        out_specs=pl.BlockSpec((1, 1, _TN), lambda i: (i, 0, 0)),
        out_shape=jax.ShapeDtypeStruct((_NB, 1, _TN), jnp.int32),
        scratch_shapes=[pltpu.VMEM((_VOCAB, _C), jnp.float32)],
    )(f_nat, emb)


_NW = 32
_NS = 16
_BPW = _N // _NW
_CHUNK = 96
_NCH = _BPW // _CHUNK
_FW = 16
_RPS = _VOCAB // _NS


def _sc_gather_body(idx_hbm, emb_hbm, out_hbm, used_hbm,
                    idx_v, rows_v, zb_v, ones_v, flags_v, used_sp, sem):
    cid = lax.axis_index("c")
    sid = lax.axis_index("s")
    wid = sid * 2 + cid
    base = wid * _BPW
    pltpu.sync_copy(idx_hbm.at[wid], idx_v)
    copies = [
        pltpu.async_copy(
            emb_hbm.at[idx_v.at[j]],
            rows_v.at[pl.ds(j * _CHUNK, _CHUNK)],
            sem,
        )
        for j in range(_NCH)
    ]

    z = jnp.zeros((_FW,), jnp.float32)
    for i in range(128):
        zb_v[i, :] = z
    for j in range(_RPS // 128):
        pltpu.sync_copy(zb_v, used_sp.at[pl.ds(sid * _RPS + j * 128, 128)])
    o = jnp.ones((_FW,), jnp.float32)
    for i in range(_CHUNK):
        ones_v[i, :] = o
    plsc.subcore_barrier()
    for j in range(_NCH):
        pltpu.sync_copy(ones_v, used_sp.at[idx_v.at[j]])
    plsc.subcore_barrier()
    pltpu.sync_copy(used_sp.at[pl.ds(sid * _RPS, _RPS)], flags_v)
    pltpu.sync_copy(flags_v, used_hbm.at[cid, pl.ds(sid * _RPS, _RPS)])

    for c in copies:
        c.wait()
    pltpu.sync_copy(rows_v, out_hbm.at[pl.ds(base, _BPW)])


_stage2 = functools.partial(
    pl.kernel,
    mesh=plsc.VectorSubcoreMesh(core_axis_name="c", subcore_axis_name="s"),
    compiler_params=pltpu.CompilerParams(use_tc_tiling_on_sc=False),
    out_type=[
        jax.ShapeDtypeStruct((_N, _C), jnp.float32),
        jax.ShapeDtypeStruct((2, _VOCAB, _FW), jnp.float32),
    ],
    scratch_types=[
        pltpu.VMEM((_NCH, _CHUNK), jnp.int32),
        pltpu.VMEM((_BPW, _C), jnp.float32),
        pltpu.VMEM((128, _FW), jnp.float32),
        pltpu.VMEM((_CHUNK, _FW), jnp.float32),
        pltpu.VMEM((_RPS, _FW), jnp.float32),
        pltpu.VMEM_SHARED((_VOCAB, _FW), jnp.float32),
        pltpu.SemaphoreType.DMA,
    ],
)(_sc_gather_body)


def _s3_body(h_ref, f_ref, wt_ref, b_ref, used_ref,
             fhat_ref, loss_ref, usage_ref, pad_ref, col_ref, sse_ref):
    b = pl.program_id(0)
    pad_ref[...] = jnp.zeros_like(pad_ref)
    pad_ref[1:_T + 1, 1:_H + 1, 1:_W + 1, :] = h_ref[...].reshape(
        _T, _H, _W, _C)
    k = 0
    for dt in (0, 1, 2):
        for dh in (0, 1, 2):
            for dw in (0, 1, 2):
                col_ref[:, k * _C:(k + 1) * _C] = (
                    pad_ref[dt:dt + _T, dh:dh + _H, dw:dw + _W, :]
                    .reshape(_TOK, _C))
                k += 1
    acc = jnp.dot(col_ref[...].astype(jnp.bfloat16), wt_ref[...],
                  preferred_element_type=jnp.float32)
    x = h_ref[...].reshape(_TOK, _C)
    fhat = (1.0 - _RESI) * x + _RESI * (acc + b_ref[...])
    fhat_ref[...] = fhat.T.reshape(1, _C, _TOK)
    d = fhat - f_ref[0].T
    part = jnp.sum(d * d)

    @pl.when(b == 0)
    def _z():
        sse_ref[0] = 0.0

    sse_ref[0] += part

    @pl.when(b == _B - 1)
    def _fin():
        loss_ref[...] = ((1.0 + _BETA) * sse_ref[0] / float(_N * _C)).reshape(1, 1)
        u = used_ref[...]
        half = u.shape[0] // 2
        any16 = jnp.maximum(u[:half], u[half:])
        cnt = jnp.sum((any16 > 0.5).astype(jnp.float32)) / float(_FW)
        usage_ref[...] = (cnt / float(_VOCAB) * 100.0).reshape(1, 1)


def _stage3(h_tok, f_tok, wtflat, bias, used):
    return pl.pallas_call(
        _s3_body,
        grid=(_B,),
        in_specs=[
            pl.BlockSpec((1, _TOK, _C), lambda b: (b, 0, 0)),
            pl.BlockSpec((1, _C, _TOK), lambda b: (b, 0, 0)),
            pl.BlockSpec((27 * _C, _C), lambda b: (0, 0)),
            pl.BlockSpec((1, _C), lambda b: (0, 0)),
            pl.BlockSpec((2 * _VOCAB * _FW // 128, 128), lambda b: (0, 0)),
        ],
        out_specs=[
            pl.BlockSpec((1, _C, _TOK), lambda b: (b, 0, 0)),
            pl.BlockSpec((1, 1), lambda b: (0, 0)),
            pl.BlockSpec((1, 1), lambda b: (0, 0)),
        ],
        out_shape=[
            jax.ShapeDtypeStruct((_B, _C, _TOK), jnp.float32),
            jax.ShapeDtypeStruct((1, 1), jnp.float32),
            jax.ShapeDtypeStruct((1, 1), jnp.float32),
        ],
        scratch_shapes=[
            pltpu.VMEM((_T + 2, _H + 2, _W + 2, _C), jnp.float32),
            pltpu.VMEM((_TOK, 27 * _C), jnp.float32),
            pltpu.SMEM((1,), jnp.float32),
        ],
    )(h_tok, f_tok, wtflat, bias, used)


def kernel(f_BCTHW, emb_weight, conv_w, conv_b):
    f = f_BCTHW.astype(jnp.float32)
    emb = emb_weight.astype(jnp.float32)
    f_nat = f.reshape(_B, _C, _TOK)
    wtflat = conv_w.astype(jnp.float32).transpose(2, 3, 4, 1, 0).reshape(
        27 * _C, _C).astype(jnp.bfloat16)
    bias = conv_b.astype(jnp.float32).reshape(1, _C)

    idx_blocks = _stage1(f_nat, emb)
    idx_grp = idx_blocks.reshape(_NW, _NCH, _CHUNK)
    h, used = _stage2(idx_grp, emb)
    h_tok = h.reshape(_B, _TOK, _C)
    used_flat = used.reshape(2 * _VOCAB * _FW // 128, 128)
    fhat_nat, loss, usage = _stage3(h_tok, f_nat, wtflat, bias, used_flat)
    fhat = fhat_nat.reshape(_B, _C, _T, _H, _W)
    return fhat, loss[0, 0], usage[0, 0]


# speedup vs baseline: 1.1419x; 1.1419x over previous
import functools

import jax
import jax.numpy as jnp
from jax import lax
from jax.experimental import pallas as pl
from jax.experimental.pallas import tpu as pltpu
from jax.experimental.pallas import tpu_sc as plsc

_VOCAB = 8192
_C = 64
_B, _T, _H, _W = 4, 8, 24, 24
_N = _B * _T * _H * _W
_TOK = _T * _H * _W
_TN = 512
_NB = _N // _TN
_BETA = 0.25
_RESI = 0.5


def _s1_body(q_ref, emb_ref, idx_ref, cb_ref):
    i = pl.program_id(0)

    @pl.when(i == 0)
    def _init():
        e = emb_ref[...]
        n = jnp.sqrt(jnp.sum(e * e, axis=1, keepdims=True))
        cb_ref[...] = e / jnp.maximum(n, 1e-12)

    q = q_ref[...]
    qn = jnp.sqrt(jnp.sum(q * q, axis=1, keepdims=True))
    q = q / jnp.maximum(qn, 1e-12)
    lane = lax.broadcasted_iota(jnp.int32, (1, 128), 1).astype(jnp.float32)
    s = lax.dot_general(q, cb_ref[...], (((1,), (1,)), ((), ())),
                        preferred_element_type=jnp.float32)
    for g in range(_TN // 64):
        rows = slice(g * 64, (g + 1) * 64)
        bestv = jnp.full((64, 128), -jnp.inf, jnp.float32)
        besti = jnp.zeros((64, 128), jnp.float32)
        for j in range(_VOCAB // 128):
            blk = s[rows, j * 128:(j + 1) * 128]
            gt = blk > bestv
            besti = jnp.where(gt, float(j), besti)
            bestv = jnp.maximum(blk, bestv)
        m = jnp.max(bestv, axis=1, keepdims=True)
        cand = jnp.where(bestv == m, besti * 128.0 + lane, 1e9)
        idx = jnp.min(cand, axis=1)
        idx_ref[0, 0, rows] = idx.astype(jnp.int32)


def _stage1(q_NxC, emb):
    return pl.pallas_call(
        _s1_body,
        grid=(_NB,),
        in_specs=[
            pl.BlockSpec((_TN, _C), lambda i: (i, 0)),
            pl.BlockSpec((_VOCAB, _C), lambda i: (0, 0)),
        ],
        out_specs=pl.BlockSpec((1, 1, _TN), lambda i: (i, 0, 0)),
        out_shape=jax.ShapeDtypeStruct((_NB, 1, _TN), jnp.int32),
        scratch_shapes=[pltpu.VMEM((_VOCAB, _C), jnp.float32)],
    )(q_NxC, emb)


_NW = 32
_NS = 16
_BPW = _N // _NW
_CHUNK = 96
_NCH = _BPW // _CHUNK
_FW = 16
_RPS = _VOCAB // _NS


def _sc_gather_body(idx_hbm, emb_hbm, out_hbm, used_hbm,
                    idx_v, rows_v, zb_v, ones_v, flags_v, used_sp, sem):
    cid = lax.axis_index("c")
    sid = lax.axis_index("s")
    wid = sid * 2 + cid
    base = wid * _BPW
    pltpu.sync_copy(idx_hbm.at[wid], idx_v)
    copies = [
        pltpu.async_copy(
            emb_hbm.at[idx_v.at[j]],
            rows_v.at[pl.ds(j * _CHUNK, _CHUNK)],
            sem,
        )
        for j in range(_NCH)
    ]

    z = jnp.zeros((_FW,), jnp.float32)
    for i in range(128):
        zb_v[i, :] = z
    for j in range(_RPS // 128):
        pltpu.sync_copy(zb_v, used_sp.at[pl.ds(sid * _RPS + j * 128, 128)])
    o = jnp.ones((_FW,), jnp.float32)
    for i in range(_CHUNK):
        ones_v[i, :] = o
    plsc.subcore_barrier()
    for j in range(_NCH):
        pltpu.sync_copy(ones_v, used_sp.at[idx_v.at[j]])
    plsc.subcore_barrier()
    pltpu.sync_copy(used_sp.at[pl.ds(sid * _RPS, _RPS)], flags_v)
    pltpu.sync_copy(flags_v, used_hbm.at[cid, pl.ds(sid * _RPS, _RPS)])

    for c in copies:
        c.wait()
    pltpu.sync_copy(rows_v, out_hbm.at[pl.ds(base, _BPW)])


_stage2 = functools.partial(
    pl.kernel,
    mesh=plsc.VectorSubcoreMesh(core_axis_name="c", subcore_axis_name="s"),
    compiler_params=pltpu.CompilerParams(use_tc_tiling_on_sc=False),
    out_type=[
        jax.ShapeDtypeStruct((_N, _C), jnp.float32),
        jax.ShapeDtypeStruct((2, _VOCAB, _FW), jnp.float32),
    ],
    scratch_types=[
        pltpu.VMEM((_NCH, _CHUNK), jnp.int32),
        pltpu.VMEM((_BPW, _C), jnp.float32),
        pltpu.VMEM((128, _FW), jnp.float32),
        pltpu.VMEM((_CHUNK, _FW), jnp.float32),
        pltpu.VMEM((_RPS, _FW), jnp.float32),
        pltpu.VMEM_SHARED((_VOCAB, _FW), jnp.float32),
        pltpu.SemaphoreType.DMA,
    ],
)(_sc_gather_body)


def _s3_body(h_ref, f_ref, wt_ref, b_ref, used_ref,
             fhat_ref, loss_ref, usage_ref, pad_ref, col_ref, sse_ref):
    b = pl.program_id(0)
    pad_ref[...] = jnp.zeros_like(pad_ref)
    pad_ref[1:_T + 1, 1:_H + 1, 1:_W + 1, :] = h_ref[...].reshape(
        _T, _H, _W, _C)
    k = 0
    for dt in (0, 1, 2):
        for dh in (0, 1, 2):
            for dw in (0, 1, 2):
                col_ref[:, k * _C:(k + 1) * _C] = (
                    pad_ref[dt:dt + _T, dh:dh + _H, dw:dw + _W, :]
                    .reshape(_TOK, _C))
                k += 1
    acc = jnp.dot(col_ref[...].astype(jnp.bfloat16), wt_ref[...],
                  preferred_element_type=jnp.float32)
    x = h_ref[...].reshape(_TOK, _C)
    fhat = (1.0 - _RESI) * x + _RESI * (acc + b_ref[...])
    fhat_ref[...] = fhat.reshape(1, _TOK, _C)
    d = fhat - f_ref[...].reshape(_TOK, _C)
    part = jnp.sum(d * d)

    @pl.when(b == 0)
    def _z():
        sse_ref[0] = 0.0

    sse_ref[0] += part

    @pl.when(b == _B - 1)
    def _fin():
        loss_ref[...] = ((1.0 + _BETA) * sse_ref[0] / float(_N * _C)).reshape(1, 1)
        u = used_ref[...]
        half = u.shape[0] // 2
        any16 = jnp.maximum(u[:half], u[half:])
        cnt = jnp.sum((any16 > 0.5).astype(jnp.float32)) / float(_FW)
        usage_ref[...] = (cnt / float(_VOCAB) * 100.0).reshape(1, 1)


def _stage3(h_tok, f_tok, wtflat, bias, used):
    return pl.pallas_call(
        _s3_body,
        grid=(_B,),
        in_specs=[
            pl.BlockSpec((1, _TOK, _C), lambda b: (b, 0, 0)),
            pl.BlockSpec((1, _TOK, _C), lambda b: (b, 0, 0)),
            pl.BlockSpec((27 * _C, _C), lambda b: (0, 0)),
            pl.BlockSpec((1, _C), lambda b: (0, 0)),
            pl.BlockSpec((2 * _VOCAB * _FW // 128, 128), lambda b: (0, 0)),
        ],
        out_specs=[
            pl.BlockSpec((1, _TOK, _C), lambda b: (b, 0, 0)),
            pl.BlockSpec((1, 1), lambda b: (0, 0)),
            pl.BlockSpec((1, 1), lambda b: (0, 0)),
        ],
        out_shape=[
            jax.ShapeDtypeStruct((_B, _TOK, _C), jnp.float32),
            jax.ShapeDtypeStruct((1, 1), jnp.float32),
            jax.ShapeDtypeStruct((1, 1), jnp.float32),
        ],
        scratch_shapes=[
            pltpu.VMEM((_T + 2, _H + 2, _W + 2, _C), jnp.float32),
            pltpu.VMEM((_TOK, 27 * _C), jnp.float32),
            pltpu.SMEM((1,), jnp.float32),
        ],
    )(h_tok, f_tok, wtflat, bias, used)


def kernel(f_BCTHW, emb_weight, conv_w, conv_b):
    f = f_BCTHW.astype(jnp.float32)
    emb = emb_weight.astype(jnp.float32)
    f_tok = f.transpose(0, 2, 3, 4, 1).reshape(_B, _TOK, _C)
    q = f_tok.reshape(_N, _C)
    wtflat = conv_w.astype(jnp.float32).transpose(2, 3, 4, 1, 0).reshape(
        27 * _C, _C).astype(jnp.bfloat16)
    bias = conv_b.astype(jnp.float32).reshape(1, _C)

    idx_blocks = _stage1(q, emb)
    idx_grp = idx_blocks.reshape(_NW, _NCH, _CHUNK)
    h, used = _stage2(idx_grp, emb)
    h_tok = h.reshape(_B, _TOK, _C)
    used_flat = used.reshape(2 * _VOCAB * _FW // 128, 128)
    fhat_tok, loss, usage = _stage3(h_tok, f_tok, wtflat, bias, used_flat)
    fhat = fhat_tok.reshape(_B, _T, _H, _W, _C).transpose(0, 4, 1, 2, 3)
    return fhat, loss[0, 0], usage[0, 0]


# speedup vs baseline: 1.1549x; 1.0114x over previous
import functools

import jax
import jax.numpy as jnp
from jax import lax
from jax.experimental import pallas as pl
from jax.experimental.pallas import tpu as pltpu
from jax.experimental.pallas import tpu_sc as plsc

_VOCAB = 8192
_C = 64
_B, _T, _H, _W = 4, 8, 24, 24
_N = _B * _T * _H * _W
_TOK = _T * _H * _W
_TN = 512
_NB = _N // _TN
_BETA = 0.25
_RESI = 0.5


def _s1_body(q_ref, emb_ref, idx_ref, cb_ref):
    i = pl.program_id(0)

    @pl.when(i == 0)
    def _init():
        e = emb_ref[...]
        n = jnp.sqrt(jnp.sum(e * e, axis=1, keepdims=True))
        cb_ref[...] = e / jnp.maximum(n, 1e-12)

    q = q_ref[...]
    qn = jnp.sqrt(jnp.sum(q * q, axis=1, keepdims=True))
    q = q / jnp.maximum(qn, 1e-12)
    lane = lax.broadcasted_iota(jnp.int32, (1, 128), 1).astype(jnp.float32)
    halves = [
        lax.dot_general(q[h * (_TN // 2):(h + 1) * (_TN // 2), :],
                        cb_ref[...], (((1,), (1,)), ((), ())),
                        preferred_element_type=jnp.float32)
        for h in range(2)
    ]
    for g in range(_TN // 64):
        rows = slice(g * 64, (g + 1) * 64)
        s = halves[g // (_TN // 128)]
        srows = slice((g * 64) % (_TN // 2), (g * 64) % (_TN // 2) + 64)
        bestv = jnp.full((64, 128), -jnp.inf, jnp.float32)
        besti = jnp.zeros((64, 128), jnp.float32)
        for j in range(_VOCAB // 128):
            blk = s[srows, j * 128:(j + 1) * 128]
            gt = blk > bestv
            besti = jnp.where(gt, float(j), besti)
            bestv = jnp.maximum(blk, bestv)
        m = jnp.max(bestv, axis=1, keepdims=True)
        cand = jnp.where(bestv == m, besti * 128.0 + lane, 1e9)
        idx = jnp.min(cand, axis=1)
        idx_ref[0, 0, rows] = idx.astype(jnp.int32)


def _stage1(q_NxC, emb):
    return pl.pallas_call(
        _s1_body,
        grid=(_NB,),
        in_specs=[
            pl.BlockSpec((_TN, _C), lambda i: (i, 0)),
            pl.BlockSpec((_VOCAB, _C), lambda i: (0, 0)),
        ],
        out_specs=pl.BlockSpec((1, 1, _TN), lambda i: (i, 0, 0)),
        out_shape=jax.ShapeDtypeStruct((_NB, 1, _TN), jnp.int32),
        scratch_shapes=[pltpu.VMEM((_VOCAB, _C), jnp.float32)],
    )(q_NxC, emb)


_NW = 32
_NS = 16
_BPW = _N // _NW
_CHUNK = 96
_NCH = _BPW // _CHUNK
_FW = 16
_RPS = _VOCAB // _NS


def _sc_gather_body(idx_hbm, emb_hbm, out_hbm, used_hbm,
                    idx_v, rows_v, zb_v, ones_v, flags_v, used_sp, sem):
    cid = lax.axis_index("c")
    sid = lax.axis_index("s")
    wid = sid * 2 + cid
    base = wid * _BPW
    pltpu.sync_copy(idx_hbm.at[wid], idx_v)
    copies = [
        pltpu.async_copy(
            emb_hbm.at[idx_v.at[j]],
            rows_v.at[pl.ds(j * _CHUNK, _CHUNK)],
            sem,
        )
        for j in range(_NCH)
    ]

    z = jnp.zeros((_FW,), jnp.float32)
    for i in range(128):
        zb_v[i, :] = z
    for j in range(_RPS // 128):
        pltpu.sync_copy(zb_v, used_sp.at[pl.ds(sid * _RPS + j * 128, 128)])
    o = jnp.ones((_FW,), jnp.float32)
    for i in range(_CHUNK):
        ones_v[i, :] = o
    plsc.subcore_barrier()
    for j in range(_NCH):
        pltpu.sync_copy(ones_v, used_sp.at[idx_v.at[j]])
    plsc.subcore_barrier()
    pltpu.sync_copy(used_sp.at[pl.ds(sid * _RPS, _RPS)], flags_v)
    pltpu.sync_copy(flags_v, used_hbm.at[cid, pl.ds(sid * _RPS, _RPS)])

    for c in copies:
        c.wait()
    pltpu.sync_copy(rows_v, out_hbm.at[pl.ds(base, _BPW)])


_stage2 = functools.partial(
    pl.kernel,
    mesh=plsc.VectorSubcoreMesh(core_axis_name="c", subcore_axis_name="s"),
    compiler_params=pltpu.CompilerParams(use_tc_tiling_on_sc=False),
    out_type=[
        jax.ShapeDtypeStruct((_N, _C), jnp.float32),
        jax.ShapeDtypeStruct((2, _VOCAB, _FW), jnp.float32),
    ],
    scratch_types=[
        pltpu.VMEM((_NCH, _CHUNK), jnp.int32),
        pltpu.VMEM((_BPW, _C), jnp.float32),
        pltpu.VMEM((128, _FW), jnp.float32),
        pltpu.VMEM((_CHUNK, _FW), jnp.float32),
        pltpu.VMEM((_RPS, _FW), jnp.float32),
        pltpu.VMEM_SHARED((_VOCAB, _FW), jnp.float32),
        pltpu.SemaphoreType.DMA,
    ],
)(_sc_gather_body)


def _s3_body(h_ref, f_ref, wt_ref, b_ref, used_ref,
             fhat_ref, loss_ref, usage_ref, pad_ref, col_ref, sse_ref):
    b = pl.program_id(0)
    pad_ref[...] = jnp.zeros_like(pad_ref)
    pad_ref[1:_T + 1, 1:_H + 1, 1:_W + 1, :] = h_ref[...].reshape(
        _T, _H, _W, _C)
    k = 0
    for dt in (0, 1, 2):
        for dh in (0, 1, 2):
            for dw in (0, 1, 2):
                col_ref[:, k * _C:(k + 1) * _C] = (
                    pad_ref[dt:dt + _T, dh:dh + _H, dw:dw + _W, :]
                    .reshape(_TOK, _C))
                k += 1
    acc = jnp.dot(col_ref[...].astype(jnp.bfloat16), wt_ref[...],
                  preferred_element_type=jnp.float32)
    x = h_ref[...].reshape(_TOK, _C)
    fhat = (1.0 - _RESI) * x + _RESI * (acc + b_ref[...])
    fhat_ref[...] = fhat.reshape(1, _TOK, _C)
    d = fhat - f_ref[...].reshape(_TOK, _C)
    part = jnp.sum(d * d)

    @pl.when(b == 0)
    def _z():
        sse_ref[0] = 0.0

    sse_ref[0] += part

    @pl.when(b == _B - 1)
    def _fin():
        loss_ref[...] = ((1.0 + _BETA) * sse_ref[0] / float(_N * _C)).reshape(1, 1)
        u = used_ref[...]
        half = u.shape[0] // 2
        any16 = jnp.maximum(u[:half], u[half:])
        cnt = jnp.sum((any16 > 0.5).astype(jnp.float32)) / float(_FW)
        usage_ref[...] = (cnt / float(_VOCAB) * 100.0).reshape(1, 1)


def _stage3(h_tok, f_tok, wtflat, bias, used):
    return pl.pallas_call(
        _s3_body,
        grid=(_B,),
        in_specs=[
            pl.BlockSpec((1, _TOK, _C), lambda b: (b, 0, 0)),
            pl.BlockSpec((1, _TOK, _C), lambda b: (b, 0, 0)),
            pl.BlockSpec((27 * _C, _C), lambda b: (0, 0)),
            pl.BlockSpec((1, _C), lambda b: (0, 0)),
            pl.BlockSpec((2 * _VOCAB * _FW // 128, 128), lambda b: (0, 0)),
        ],
        out_specs=[
            pl.BlockSpec((1, _TOK, _C), lambda b: (b, 0, 0)),
            pl.BlockSpec((1, 1), lambda b: (0, 0)),
            pl.BlockSpec((1, 1), lambda b: (0, 0)),
        ],
        out_shape=[
            jax.ShapeDtypeStruct((_B, _TOK, _C), jnp.float32),
            jax.ShapeDtypeStruct((1, 1), jnp.float32),
            jax.ShapeDtypeStruct((1, 1), jnp.float32),
        ],
        scratch_shapes=[
            pltpu.VMEM((_T + 2, _H + 2, _W + 2, _C), jnp.float32),
            pltpu.VMEM((_TOK, 27 * _C), jnp.float32),
            pltpu.SMEM((1,), jnp.float32),
        ],
    )(h_tok, f_tok, wtflat, bias, used)


def kernel(f_BCTHW, emb_weight, conv_w, conv_b):
    f = f_BCTHW.astype(jnp.float32)
    emb = emb_weight.astype(jnp.float32)
    f_tok = f.transpose(0, 2, 3, 4, 1).reshape(_B, _TOK, _C)
    q = f_tok.reshape(_N, _C)
    wtflat = conv_w.astype(jnp.float32).transpose(2, 3, 4, 1, 0).reshape(
        27 * _C, _C).astype(jnp.bfloat16)
    bias = conv_b.astype(jnp.float32).reshape(1, _C)

    idx_blocks = _stage1(q, emb)
    idx_grp = idx_blocks.reshape(_NW, _NCH, _CHUNK)
    h, used = _stage2(idx_grp, emb)
    h_tok = h.reshape(_B, _TOK, _C)
    used_flat = used.reshape(2 * _VOCAB * _FW // 128, 128)
    fhat_tok, loss, usage = _stage3(h_tok, f_tok, wtflat, bias, used_flat)
    fhat = fhat_tok.reshape(_B, _T, _H, _W, _C).transpose(0, 4, 1, 2, 3)
    return fhat, loss[0, 0], usage[0, 0]
